# baseline (speedup 1.0000x reference)
import jax
import jax.numpy as jnp
from jax.experimental import pallas as pl
from jax.experimental.pallas import tpu as pltpu


def _lerp_body(a_ref, c_ref, g_ref, o_ref):
    a = a_ref[0, 0, 0]
    c = c_ref[...]
    o_ref[...] = c + a * (g_ref[...] - c)


def kernel(x):
    B, C, W, H = x.shape
    N = W * H
    kp = jax.random.fold_in(jax.random.key(42), 1)
    kl = jax.random.fold_in(jax.random.key(42), 2)
    perm = jax.random.permutation(kp, B)
    lmda = jax.random.beta(kl, 0.1, 0.1, (B, 1, 1)).astype(x.dtype)
    content = x.reshape(B, C, N)
    style = x[perm].reshape(B, C, N)
    value_style = jnp.sort(style, axis=-1)
    iota = jax.lax.broadcasted_iota(jnp.int32, (B, C, N), 2)
    sorted_content, idx = jax.lax.sort((content, iota), dimension=2, num_keys=1)

    a = jnp.broadcast_to(1.0 - lmda, (B, C, 1)).reshape(B * C, 1, 1)
    sc = sorted_content.reshape(B * C, 1, N)
    vs = value_style.reshape(B * C, 1, N)

    merged = pl.pallas_call(
        _lerp_body,
        grid=(B * C,),
        in_specs=[
            pl.BlockSpec((1, 1, 1), lambda i: (i, 0, 0), memory_space=pltpu.SMEM),
            pl.BlockSpec((1, 1, N), lambda i: (i, 0, 0)),
            pl.BlockSpec((1, 1, N), lambda i: (i, 0, 0)),
        ],
        out_specs=pl.BlockSpec((1, 1, N), lambda i: (i, 0, 0)),
        out_shape=jax.ShapeDtypeStruct((B * C, 1, N), x.dtype),
    )(a, sc, vs)

    merged = merged.reshape(B, C, N)
    out = jnp.zeros((B, C, N), x.dtype).at[
        jnp.arange(B)[:, None, None], jnp.arange(C)[None, :, None], idx
    ].set(merged)
    return out.reshape(B, C, W, H)
